# Initial kernel scaffold; baseline (speedup 1.0000x reference)
import jax
import jax.numpy as jnp
from jax.experimental import pallas as pl


def kernel(x_user, x_item, edge_index_ui, edge_index_iu, proj_user_W, proj_user_b, proj_item_W, proj_item_b, conv1_ui_Wl, conv1_ui_bl, conv1_ui_Wr, conv1_iu_Wl, conv1_iu_bl, conv1_iu_Wr, conv2_ui_Wl, conv2_ui_bl, conv2_ui_Wr, conv2_iu_Wl, conv2_iu_bl, conv2_iu_Wr, pred_W1, pred_b1, pred_W2, pred_b2):
    raise NotImplementedError("write your pallas kernel here")


# speedup vs baseline: 4.0533x; 4.0533x over previous
import functools

import jax
import jax.numpy as jnp
from jax import lax
from jax.experimental import pallas as pl
from jax.experimental.pallas import tpu as pltpu
from jax.experimental.pallas import tpu_sc as plsc

N = 50000
E = 800000
DIN = 128
DH = 64
DO = 32
NC = 2
NS = 16
EP = 819200
PAD = EP - E
NACC = 50176
RPT = NACC // NS
K = 512
NBLK = EP // (NS * K)
IR = K // 128
ER = EP // 128
TR = NBLK * IR

_MESH = plsc.VectorSubcoreMesh(
    core_axis_name="c", subcore_axis_name="s", num_cores=NC, num_subcores=NS)


@functools.partial(
    pl.kernel,
    out_type=jax.ShapeDtypeStruct((NC * NACC, 32), jnp.float32),
    mesh=_MESH,
    scratch_types=[
        pltpu.VMEM((IR, 128), jnp.int32),
        pltpu.VMEM((IR, 128), jnp.int32),
        pltpu.VMEM((K, 32), jnp.float32),
        pltpu.VMEM_SHARED((NACC, 32), jnp.float32),
        pltpu.SemaphoreType.DMA,
    ],
    compiler_params=pltpu.CompilerParams(use_tc_tiling_on_sc=False),
)
def _sc_segsum(table, src2, dst2, zblk, out, sidx, didx, rows, acc, gsem):
    c = lax.axis_index("c")
    s = lax.axis_index("s")
    pltpu.sync_copy(zblk, acc.at[pl.ds(s * RPT, RPT)])
    plsc.subcore_barrier()

    def body(b, carry):
        r_src = c * ER + s * TR + b * IR
        r_dst = s * TR + b * IR
        pltpu.sync_copy(src2.at[pl.ds(r_src, IR)], sidx)
        pltpu.sync_copy(dst2.at[pl.ds(r_dst, IR)], didx)
        cps = [
            pltpu.async_copy(table.at[sidx.at[j]],
                             rows.at[pl.ds(j * 128, 128)], gsem)
            for j in range(IR)
        ]
        for cp in cps:
            cp.wait()
        for j in range(IR):
            pltpu.sync_copy(rows.at[pl.ds(j * 128, 128)],
                            acc.at[didx.at[j]], add=True)
        return carry

    lax.fori_loop(0, NBLK, body, 0)
    plsc.subcore_barrier()
    pltpu.sync_copy(acc.at[pl.ds(s * RPT, RPT)],
                    out.at[pl.ds(c * NACC + s * RPT, RPT)])


@functools.partial(
    pl.kernel,
    out_type=jax.ShapeDtypeStruct((NC * NACC, 32), jnp.float32),
    mesh=_MESH,
    scratch_types=[
        pltpu.VMEM((IR, 128), jnp.int32),
        pltpu.VMEM((K, 32), jnp.float32),
        pltpu.VMEM_SHARED((NACC, 32), jnp.float32),
    ],
    compiler_params=pltpu.CompilerParams(use_tc_tiling_on_sc=False),
)
def _sc_counts(dsts, oblk, zblk, out, didx, ones, acc):
    c = lax.axis_index("c")
    s = lax.axis_index("s")
    pltpu.sync_copy(oblk, ones)
    pltpu.sync_copy(zblk, acc.at[pl.ds(s * RPT, RPT)])
    plsc.subcore_barrier()

    def body(b, carry):
        r_dst = c * ER + s * TR + b * IR
        pltpu.sync_copy(dsts.at[pl.ds(r_dst, IR)], didx)
        for j in range(IR):
            pltpu.sync_copy(ones.at[pl.ds(j * 128, 128)],
                            acc.at[didx.at[j]], add=True)
        return carry

    lax.fori_loop(0, NBLK, body, 0)
    plsc.subcore_barrier()
    pltpu.sync_copy(acc.at[pl.ds(s * RPT, RPT)],
                    out.at[pl.ds(c * NACC + s * RPT, RPT)])


R = 2000
G = N // R


def _safe(h):
    h = jnp.nan_to_num(h, nan=0.0, posinf=1.0, neginf=-1.0)
    return jnp.clip(h, -10.0, 10.0)


def _leaky(h):
    return jnp.where(h >= 0, h, 0.1 * h)


def _proj_body(x_ref, w_ref, b_ref, o_ref):
    x = jnp.nan_to_num(x_ref[...])
    h = jnp.dot(x, w_ref[...], preferred_element_type=jnp.float32) + b_ref[...]
    h = _safe(h)
    o_ref[0] = h[:, :32]
    o_ref[1] = h[:, 32:]


def _proj(x, w, b):
    return pl.pallas_call(
        _proj_body,
        grid=(G,),
        in_specs=[
            pl.BlockSpec((R, DIN), lambda i: (i, 0)),
            pl.BlockSpec((DIN, DH), lambda i: (0, 0)),
            pl.BlockSpec((1, DH), lambda i: (0, 0)),
        ],
        out_specs=pl.BlockSpec((2, R, 32), lambda i: (0, i, 0)),
        out_shape=jax.ShapeDtypeStruct((2, N, 32), jnp.float32),
    )(x, w, b.reshape(1, DH))


def _combine_core(s_ref, c_ref, h_ref, w_ref, b_ref):
    cnt = c_ref[...]
    inv = 1.0 / jnp.maximum(cnt[:, 0:1], 1.0)
    mean = jnp.concatenate([s_ref[0], s_ref[1]], axis=1) * inv
    xd = jnp.concatenate([h_ref[0], h_ref[1]], axis=1)
    z = jnp.concatenate([mean, xd], axis=1)
    h = jnp.dot(z, w_ref[...], preferred_element_type=jnp.float32) + b_ref[...]
    return _leaky(_safe(h))


def _combine1_body(s_ref, c_ref, h_ref, w_ref, b_ref, o_ref):
    h = _combine_core(s_ref, c_ref, h_ref, w_ref, b_ref)
    o_ref[0] = h[:, :32]
    o_ref[1] = h[:, 32:]


def _combine1(S, cnt, h, wful, bl):
    return pl.pallas_call(
        _combine1_body,
        grid=(G,),
        in_specs=[
            pl.BlockSpec((2, R, 32), lambda i: (0, i, 0)),
            pl.BlockSpec((R, 32), lambda i: (i, 0)),
            pl.BlockSpec((2, R, 32), lambda i: (0, i, 0)),
            pl.BlockSpec((2 * DH, DH), lambda i: (0, 0)),
            pl.BlockSpec((1, DH), lambda i: (0, 0)),
        ],
        out_specs=pl.BlockSpec((2, R, 32), lambda i: (0, i, 0)),
        out_shape=jax.ShapeDtypeStruct((2, N, 32), jnp.float32),
    )(S, cnt, h, wful, bl.reshape(1, DH))


def _combine2_body(s_ref, c_ref, h_ref, w_ref, b_ref, o_ref):
    o_ref[...] = _combine_core(s_ref, c_ref, h_ref, w_ref, b_ref)


def _combine2(S, cnt, h, wful, bl):
    return pl.pallas_call(
        _combine2_body,
        grid=(G,),
        in_specs=[
            pl.BlockSpec((2, R, 32), lambda i: (0, i, 0)),
            pl.BlockSpec((R, 32), lambda i: (i, 0)),
            pl.BlockSpec((2, R, 32), lambda i: (0, i, 0)),
            pl.BlockSpec((2 * DH, DO), lambda i: (0, 0)),
            pl.BlockSpec((1, DO), lambda i: (0, 0)),
        ],
        out_specs=pl.BlockSpec((R, DO), lambda i: (i, 0)),
        out_shape=jax.ShapeDtypeStruct((N, DO), jnp.float32),
    )(S, cnt, h, wful, bl.reshape(1, DO))


def _user_body(s_ref, c_ref, h_ref, w_ref, b_ref, w1_ref, b1_ref, w2_ref,
               b2_ref, o_ref, p_ref):
    h = _combine_core(s_ref, c_ref, h_ref, w_ref, b_ref)
    o_ref[...] = h
    z = _leaky(jnp.dot(h, w1_ref[...], preferred_element_type=jnp.float32)
               + b1_ref[...])
    p = jnp.dot(z, w2_ref[...], preferred_element_type=jnp.float32) + b2_ref[...]
    p_ref[...] = jax.nn.sigmoid(p)


def _combine2_user(S, cnt, h, wful, bl, w1, b1, w2, b2):
    return pl.pallas_call(
        _user_body,
        grid=(G,),
        in_specs=[
            pl.BlockSpec((2, R, 32), lambda i: (0, i, 0)),
            pl.BlockSpec((R, 32), lambda i: (i, 0)),
            pl.BlockSpec((2, R, 32), lambda i: (0, i, 0)),
            pl.BlockSpec((2 * DH, DO), lambda i: (0, 0)),
            pl.BlockSpec((1, DO), lambda i: (0, 0)),
            pl.BlockSpec((DO, 16), lambda i: (0, 0)),
            pl.BlockSpec((1, 16), lambda i: (0, 0)),
            pl.BlockSpec((16, 1), lambda i: (0, 0)),
            pl.BlockSpec((1, 1), lambda i: (0, 0)),
        ],
        out_specs=[
            pl.BlockSpec((R, DO), lambda i: (i, 0)),
            pl.BlockSpec((R, 1), lambda i: (i, 0)),
        ],
        out_shape=[
            jax.ShapeDtypeStruct((N, DO), jnp.float32),
            jax.ShapeDtypeStruct((N, 1), jnp.float32),
        ],
    )(S, cnt, h, wful, bl.reshape(1, DO), w1, b1.reshape(1, 16), w2,
      b2.reshape(1, 1))


def _edge_arrays(edge_index):
    ei = edge_index.astype(jnp.int32)
    src = jnp.concatenate([ei[0], jnp.zeros((PAD,), jnp.int32)])
    dst = jnp.concatenate([ei[1], jnp.full((PAD,), N, jnp.int32)])
    src2 = jnp.concatenate([src, src + N]).reshape(2 * ER, 128)
    return src2, dst.reshape(ER, 128), dst


def kernel(x_user, x_item, edge_index_ui, edge_index_iu,
           proj_user_W, proj_user_b, proj_item_W, proj_item_b,
           conv1_ui_Wl, conv1_ui_bl, conv1_ui_Wr,
           conv1_iu_Wl, conv1_iu_bl, conv1_iu_Wr,
           conv2_ui_Wl, conv2_ui_bl, conv2_ui_Wr,
           conv2_iu_Wl, conv2_iu_bl, conv2_iu_Wr,
           pred_W1, pred_b1, pred_W2, pred_b2):
    src2_ui, dst2_ui, dstp_ui = _edge_arrays(edge_index_ui)
    src2_iu, dst2_iu, dstp_iu = _edge_arrays(edge_index_iu)
    dst_all = jnp.concatenate([dstp_ui, dstp_iu]).reshape(2 * ER, 128)
    zblk = jnp.zeros((RPT, 32), jnp.float32)
    oblk = jnp.ones((K, 32), jnp.float32)

    counts = _sc_counts(dst_all, oblk, zblk).reshape(2, NACC, 32)
    cnt_item = counts[0]
    cnt_user = counts[1]

    hu = _proj(x_user, proj_user_W, proj_user_b)
    hi = _proj(x_item, proj_item_W, proj_item_b)

    s_ui = _sc_segsum(hu.reshape(2 * N, 32), src2_ui, dst2_ui, zblk)
    s_iu = _sc_segsum(hi.reshape(2 * N, 32), src2_iu, dst2_iu, zblk)

    w1_ui = jnp.concatenate([conv1_ui_Wl, conv1_ui_Wr], axis=0)
    w1_iu = jnp.concatenate([conv1_iu_Wl, conv1_iu_Wr], axis=0)
    hi1 = _combine1(s_ui.reshape(2, NACC, 32), cnt_item, hi, w1_ui, conv1_ui_bl)
    hu1 = _combine1(s_iu.reshape(2, NACC, 32), cnt_user, hu, w1_iu, conv1_iu_bl)

    s2_ui = _sc_segsum(hu1.reshape(2 * N, 32), src2_ui, dst2_ui, zblk)
    s2_iu = _sc_segsum(hi1.reshape(2 * N, 32), src2_iu, dst2_iu, zblk)

    w2_ui = jnp.concatenate([conv2_ui_Wl, conv2_ui_Wr], axis=0)
    w2_iu = jnp.concatenate([conv2_iu_Wl, conv2_iu_Wr], axis=0)
    hi2 = _combine2(s2_ui.reshape(2, NACC, 32), cnt_item, hi1, w2_ui,
                    conv2_ui_bl)
    hu2, pred = _combine2_user(s2_iu.reshape(2, NACC, 32), cnt_user, hu1,
                               w2_iu, conv2_iu_bl, pred_W1, pred_b1,
                               pred_W2, pred_b2)
    return pred[:, 0], {"user": hu2, "item": hi2}


# speedup vs baseline: 4.8219x; 1.1896x over previous
import functools

import jax
import jax.numpy as jnp
from jax import lax
from jax.experimental import pallas as pl
from jax.experimental.pallas import tpu as pltpu
from jax.experimental.pallas import tpu_sc as plsc

N = 50000
E = 800000
DIN = 128
DH = 64
DO = 32
NC = 2
NS = 16
EP = 819200
PAD = EP - E
NACC = 50176
RPT = NACC // NS
K = 256
NBLK = EP // (NS * K)
IR = K // 128
ER = EP // 128
TR = NBLK * IR
SUP = 20
NSUP = NBLK // SUP
CR = SUP * IR

_MESH = plsc.VectorSubcoreMesh(
    core_axis_name="c", subcore_axis_name="s", num_cores=NC, num_subcores=NS)


@functools.partial(
    pl.kernel,
    out_type=jax.ShapeDtypeStruct((NC * NACC, 32), jnp.float32),
    mesh=_MESH,
    scratch_types=[
        pltpu.VMEM((CR, 128), jnp.int32),
        pltpu.VMEM((CR, 128), jnp.int32),
        pltpu.VMEM((K, 32), jnp.float32),
        pltpu.VMEM((K, 32), jnp.float32),
        pltpu.VMEM_SHARED((NACC, 32), jnp.float32),
        pltpu.SemaphoreType.DMA,
        pltpu.SemaphoreType.DMA,
        pltpu.SemaphoreType.DMA,
        pltpu.SemaphoreType.DMA,
    ],
    compiler_params=pltpu.CompilerParams(use_tc_tiling_on_sc=False),
)
def _sc_segsum(table, src2, dst2, zblk, out, sidx, didx, rows0, rows1, acc,
               g0, g1, s0, s1):
    c = lax.axis_index("c")
    s = lax.axis_index("s")
    pltpu.sync_copy(zblk, acc.at[pl.ds(s * RPT, RPT)])
    plsc.subcore_barrier()
    rows = (rows0, rows1)
    gsem = (g0, g1)
    ssem = (s0, s1)

    def gfire(sb, p):
        for j in range(IR):
            pltpu.async_copy(table.at[sidx.at[sb * IR + j]],
                             rows[p].at[pl.ds(j * 128, 128)], gsem[p])

    def gdrain(sb, p):
        for j in range(IR):
            pltpu.make_async_copy(table.at[sidx.at[sb * IR + j]],
                                  rows[p].at[pl.ds(j * 128, 128)],
                                  gsem[p]).wait()

    def sfire(sb, p):
        for j in range(IR):
            pltpu.async_copy(rows[p].at[pl.ds(j * 128, 128)],
                             acc.at[didx.at[sb * IR + j]], ssem[p], add=True)

    def sdrain(sb, p):
        for j in range(IR):
            pltpu.make_async_copy(rows[p].at[pl.ds(j * 128, 128)],
                                  acc.at[didx.at[sb * IR + j]],
                                  ssem[p]).wait()

    def chunk(ss, carry):
        r0 = s * TR + ss * CR
        pltpu.sync_copy(src2.at[pl.ds(c * ER + r0, CR)], sidx)
        pltpu.sync_copy(dst2.at[pl.ds(r0, CR)], didx)
        gfire(0, 0)
        for sb in range(SUP):
            p = sb & 1
            gdrain(sb, p)
            if sb + 1 < SUP:
                if sb >= 1:
                    sdrain(sb - 1, 1 - p)
                gfire(sb + 1, 1 - p)
            sfire(sb, p)
        sdrain(SUP - 2, (SUP - 2) & 1)
        sdrain(SUP - 1, (SUP - 1) & 1)
        return carry

    lax.fori_loop(0, NSUP, chunk, 0)
    plsc.subcore_barrier()
    pltpu.sync_copy(acc.at[pl.ds(s * RPT, RPT)],
                    out.at[pl.ds(c * NACC + s * RPT, RPT)])


CW = 16
LAG = 4


@functools.partial(
    pl.kernel,
    out_type=jax.ShapeDtypeStruct((NC * NACC, CW), jnp.float32),
    mesh=_MESH,
    scratch_types=[
        pltpu.VMEM((CR, 128), jnp.int32),
        pltpu.VMEM((128, CW), jnp.float32),
        pltpu.VMEM_SHARED((NACC, CW), jnp.float32),
        pltpu.SemaphoreType.DMA,
    ],
    compiler_params=pltpu.CompilerParams(use_tc_tiling_on_sc=False),
)
def _sc_counts(dsts, oblk, zblk, out, didx, ones, acc, sem):
    c = lax.axis_index("c")
    s = lax.axis_index("s")
    pltpu.sync_copy(oblk, ones)
    pltpu.sync_copy(zblk, acc.at[pl.ds(s * RPT, RPT)])
    plsc.subcore_barrier()

    def chunk(ss, carry):
        r0 = c * ER + s * TR + ss * CR
        pltpu.sync_copy(dsts.at[pl.ds(r0, CR)], didx)
        for r in range(CR):
            pltpu.async_copy(ones, acc.at[didx.at[r]], sem, add=True)
            if r >= LAG:
                pltpu.make_async_copy(ones, acc.at[didx.at[r - LAG]],
                                      sem).wait()
        for r in range(CR - LAG, CR):
            pltpu.make_async_copy(ones, acc.at[didx.at[r]], sem).wait()
        return carry

    lax.fori_loop(0, NSUP, chunk, 0)
    plsc.subcore_barrier()
    pltpu.sync_copy(acc.at[pl.ds(s * RPT, RPT)],
                    out.at[pl.ds(c * NACC + s * RPT, RPT)])


R = 2000
G = N // R


def _safe(h):
    h = jnp.nan_to_num(h, nan=0.0, posinf=1.0, neginf=-1.0)
    return jnp.clip(h, -10.0, 10.0)


def _leaky(h):
    return jnp.where(h >= 0, h, 0.1 * h)


def _proj_body(x_ref, w_ref, b_ref, o_ref):
    x = jnp.nan_to_num(x_ref[...])
    h = jnp.dot(x, w_ref[...], preferred_element_type=jnp.float32) + b_ref[...]
    h = _safe(h)
    o_ref[0] = h[:, :32]
    o_ref[1] = h[:, 32:]


def _proj(x, w, b):
    return pl.pallas_call(
        _proj_body,
        grid=(G,),
        in_specs=[
            pl.BlockSpec((R, DIN), lambda i: (i, 0)),
            pl.BlockSpec((DIN, DH), lambda i: (0, 0)),
            pl.BlockSpec((1, DH), lambda i: (0, 0)),
        ],
        out_specs=pl.BlockSpec((2, R, 32), lambda i: (0, i, 0)),
        out_shape=jax.ShapeDtypeStruct((2, N, 32), jnp.float32),
    )(x, w, b.reshape(1, DH))


def _combine_core(s_ref, c_ref, h_ref, w_ref, b_ref):
    cnt = c_ref[...]
    inv = 1.0 / jnp.maximum(cnt[:, 0:1], 1.0)
    mean = jnp.concatenate([s_ref[0], s_ref[1]], axis=1) * inv
    xd = jnp.concatenate([h_ref[0], h_ref[1]], axis=1)
    z = jnp.concatenate([mean, xd], axis=1)
    h = jnp.dot(z, w_ref[...], preferred_element_type=jnp.float32) + b_ref[...]
    return _leaky(_safe(h))


def _combine1_body(s_ref, c_ref, h_ref, w_ref, b_ref, o_ref):
    h = _combine_core(s_ref, c_ref, h_ref, w_ref, b_ref)
    o_ref[0] = h[:, :32]
    o_ref[1] = h[:, 32:]


def _combine1(S, cnt, h, wful, bl):
    return pl.pallas_call(
        _combine1_body,
        grid=(G,),
        in_specs=[
            pl.BlockSpec((2, R, 32), lambda i: (0, i, 0)),
            pl.BlockSpec((R, CW), lambda i: (i, 0)),
            pl.BlockSpec((2, R, 32), lambda i: (0, i, 0)),
            pl.BlockSpec((2 * DH, DH), lambda i: (0, 0)),
            pl.BlockSpec((1, DH), lambda i: (0, 0)),
        ],
        out_specs=pl.BlockSpec((2, R, 32), lambda i: (0, i, 0)),
        out_shape=jax.ShapeDtypeStruct((2, N, 32), jnp.float32),
    )(S, cnt, h, wful, bl.reshape(1, DH))


def _combine2_body(s_ref, c_ref, h_ref, w_ref, b_ref, o_ref):
    o_ref[...] = _combine_core(s_ref, c_ref, h_ref, w_ref, b_ref)


def _combine2(S, cnt, h, wful, bl):
    return pl.pallas_call(
        _combine2_body,
        grid=(G,),
        in_specs=[
            pl.BlockSpec((2, R, 32), lambda i: (0, i, 0)),
            pl.BlockSpec((R, CW), lambda i: (i, 0)),
            pl.BlockSpec((2, R, 32), lambda i: (0, i, 0)),
            pl.BlockSpec((2 * DH, DO), lambda i: (0, 0)),
            pl.BlockSpec((1, DO), lambda i: (0, 0)),
        ],
        out_specs=pl.BlockSpec((R, DO), lambda i: (i, 0)),
        out_shape=jax.ShapeDtypeStruct((N, DO), jnp.float32),
    )(S, cnt, h, wful, bl.reshape(1, DO))


def _user_body(s_ref, c_ref, h_ref, w_ref, b_ref, w1_ref, b1_ref, w2_ref,
               b2_ref, o_ref, p_ref):
    h = _combine_core(s_ref, c_ref, h_ref, w_ref, b_ref)
    o_ref[...] = h
    z = _leaky(jnp.dot(h, w1_ref[...], preferred_element_type=jnp.float32)
               + b1_ref[...])
    p = jnp.dot(z, w2_ref[...], preferred_element_type=jnp.float32) + b2_ref[...]
    p_ref[...] = jax.nn.sigmoid(p)


def _combine2_user(S, cnt, h, wful, bl, w1, b1, w2, b2):
    return pl.pallas_call(
        _user_body,
        grid=(G,),
        in_specs=[
            pl.BlockSpec((2, R, 32), lambda i: (0, i, 0)),
            pl.BlockSpec((R, CW), lambda i: (i, 0)),
            pl.BlockSpec((2, R, 32), lambda i: (0, i, 0)),
            pl.BlockSpec((2 * DH, DO), lambda i: (0, 0)),
            pl.BlockSpec((1, DO), lambda i: (0, 0)),
            pl.BlockSpec((DO, 16), lambda i: (0, 0)),
            pl.BlockSpec((1, 16), lambda i: (0, 0)),
            pl.BlockSpec((16, 1), lambda i: (0, 0)),
            pl.BlockSpec((1, 1), lambda i: (0, 0)),
        ],
        out_specs=[
            pl.BlockSpec((R, DO), lambda i: (i, 0)),
            pl.BlockSpec((R, 1), lambda i: (i, 0)),
        ],
        out_shape=[
            jax.ShapeDtypeStruct((N, DO), jnp.float32),
            jax.ShapeDtypeStruct((N, 1), jnp.float32),
        ],
    )(S, cnt, h, wful, bl.reshape(1, DO), w1, b1.reshape(1, 16), w2,
      b2.reshape(1, 1))


def _edge_arrays(edge_index):
    ei = edge_index.astype(jnp.int32)
    src = jnp.concatenate([ei[0], jnp.zeros((PAD,), jnp.int32)])
    dst = jnp.concatenate([ei[1], jnp.full((PAD,), N, jnp.int32)])
    src2 = jnp.concatenate([src, src + N]).reshape(2 * ER, 128)
    return src2, dst.reshape(ER, 128), dst


def kernel(x_user, x_item, edge_index_ui, edge_index_iu,
           proj_user_W, proj_user_b, proj_item_W, proj_item_b,
           conv1_ui_Wl, conv1_ui_bl, conv1_ui_Wr,
           conv1_iu_Wl, conv1_iu_bl, conv1_iu_Wr,
           conv2_ui_Wl, conv2_ui_bl, conv2_ui_Wr,
           conv2_iu_Wl, conv2_iu_bl, conv2_iu_Wr,
           pred_W1, pred_b1, pred_W2, pred_b2):
    src2_ui, dst2_ui, dstp_ui = _edge_arrays(edge_index_ui)
    src2_iu, dst2_iu, dstp_iu = _edge_arrays(edge_index_iu)
    dst_all = jnp.concatenate([dstp_ui, dstp_iu]).reshape(2 * ER, 128)
    zblk = jnp.zeros((RPT, 32), jnp.float32)
    zblkc = jnp.zeros((RPT, CW), jnp.float32)
    oblk = jnp.ones((128, CW), jnp.float32)

    counts = _sc_counts(dst_all, oblk, zblkc).reshape(2, NACC, CW)
    cnt_item = counts[0]
    cnt_user = counts[1]

    hu = _proj(x_user, proj_user_W, proj_user_b)
    hi = _proj(x_item, proj_item_W, proj_item_b)

    s_ui = _sc_segsum(hu.reshape(2 * N, 32), src2_ui, dst2_ui, zblk)
    s_iu = _sc_segsum(hi.reshape(2 * N, 32), src2_iu, dst2_iu, zblk)

    w1_ui = jnp.concatenate([conv1_ui_Wl, conv1_ui_Wr], axis=0)
    w1_iu = jnp.concatenate([conv1_iu_Wl, conv1_iu_Wr], axis=0)
    hi1 = _combine1(s_ui.reshape(2, NACC, 32), cnt_item, hi, w1_ui, conv1_ui_bl)
    hu1 = _combine1(s_iu.reshape(2, NACC, 32), cnt_user, hu, w1_iu, conv1_iu_bl)

    s2_ui = _sc_segsum(hu1.reshape(2 * N, 32), src2_ui, dst2_ui, zblk)
    s2_iu = _sc_segsum(hi1.reshape(2 * N, 32), src2_iu, dst2_iu, zblk)

    w2_ui = jnp.concatenate([conv2_ui_Wl, conv2_ui_Wr], axis=0)
    w2_iu = jnp.concatenate([conv2_iu_Wl, conv2_iu_Wr], axis=0)
    hi2 = _combine2(s2_ui.reshape(2, NACC, 32), cnt_item, hi1, w2_ui,
                    conv2_ui_bl)
    hu2, pred = _combine2_user(s2_iu.reshape(2, NACC, 32), cnt_user, hu1,
                               w2_iu, conv2_iu_bl, pred_W1, pred_b1,
                               pred_W2, pred_b2)
    return pred[:, 0], {"user": hu2, "item": hi2}


# speedup vs baseline: 5.2635x; 1.0916x over previous
import functools

import jax
import jax.numpy as jnp
from jax import lax
from jax.experimental import pallas as pl
from jax.experimental.pallas import tpu as pltpu
from jax.experimental.pallas import tpu_sc as plsc

N = 50000
E = 800000
DIN = 128
DH = 64
DO = 32
NC = 2
NS = 16
EP = 819200
PAD = EP - E
NACC = 50176
RPT = NACC // NS
K = 256
NBLK = EP // (NS * K)
IR = K // 128
ER = EP // 128
TR = NBLK * IR
SUP = 20
NSUP = NBLK // SUP
CR = SUP * IR

_MESH = plsc.VectorSubcoreMesh(
    core_axis_name="c", subcore_axis_name="s", num_cores=NC, num_subcores=NS)


@functools.partial(
    pl.kernel,
    out_type=jax.ShapeDtypeStruct((NC * NACC, 32), jnp.float32),
    mesh=_MESH,
    scratch_types=[
        pltpu.VMEM((CR, 128), jnp.int32),
        pltpu.VMEM((CR, 128), jnp.int32),
        pltpu.VMEM((K, 32), jnp.float32),
        pltpu.VMEM((K, 32), jnp.float32),
        pltpu.VMEM_SHARED((NACC, 32), jnp.float32),
        pltpu.SemaphoreType.DMA,
        pltpu.SemaphoreType.DMA,
        pltpu.SemaphoreType.DMA,
        pltpu.SemaphoreType.DMA,
    ],
    compiler_params=pltpu.CompilerParams(use_tc_tiling_on_sc=False),
)
def _sc_segsum(table, src2, dst2, zblk, out, sidx, didx, rows0, rows1, acc,
               g0, g1, s0, s1):
    c = lax.axis_index("c")
    s = lax.axis_index("s")
    pltpu.sync_copy(zblk, acc.at[pl.ds(s * RPT, RPT)])
    plsc.subcore_barrier()
    rows = (rows0, rows1)
    gsem = (g0, g1)
    ssem = (s0, s1)

    def gfire(sb, p):
        for j in range(IR):
            pltpu.async_copy(table.at[sidx.at[sb * IR + j]],
                             rows[p].at[pl.ds(j * 128, 128)], gsem[p])

    def gdrain(sb, p):
        for j in range(IR):
            pltpu.make_async_copy(table.at[sidx.at[sb * IR + j]],
                                  rows[p].at[pl.ds(j * 128, 128)],
                                  gsem[p]).wait()

    def sfire(sb, p):
        for j in range(IR):
            pltpu.async_copy(rows[p].at[pl.ds(j * 128, 128)],
                             acc.at[didx.at[sb * IR + j]], ssem[p], add=True)

    def sdrain(sb, p):
        for j in range(IR):
            pltpu.make_async_copy(rows[p].at[pl.ds(j * 128, 128)],
                                  acc.at[didx.at[sb * IR + j]],
                                  ssem[p]).wait()

    def chunk(ss, carry):
        r0 = s * TR + ss * CR
        pltpu.sync_copy(src2.at[pl.ds(c * ER + r0, CR)], sidx)
        pltpu.sync_copy(dst2.at[pl.ds(r0, CR)], didx)
        gfire(0, 0)
        for sb in range(SUP):
            p = sb & 1
            if sb + 1 < SUP:
                if sb >= 1:
                    sdrain(sb - 1, 1 - p)
                gfire(sb + 1, 1 - p)
            gdrain(sb, p)
            sfire(sb, p)
        sdrain(SUP - 2, (SUP - 2) & 1)
        sdrain(SUP - 1, (SUP - 1) & 1)
        return carry

    lax.fori_loop(0, NSUP, chunk, 0)
    plsc.subcore_barrier()
    pltpu.sync_copy(acc.at[pl.ds(s * RPT, RPT)],
                    out.at[pl.ds(c * NACC + s * RPT, RPT)])


CW = 16
LAG = 4


@functools.partial(
    pl.kernel,
    out_type=jax.ShapeDtypeStruct((NC * NACC, CW), jnp.float32),
    mesh=_MESH,
    scratch_types=[
        pltpu.VMEM((CR, 128), jnp.int32),
        pltpu.VMEM((128, CW), jnp.float32),
        pltpu.VMEM_SHARED((NACC, CW), jnp.float32),
        pltpu.SemaphoreType.DMA,
    ],
    compiler_params=pltpu.CompilerParams(use_tc_tiling_on_sc=False),
)
def _sc_counts(dsts, oblk, zblk, out, didx, ones, acc, sem):
    c = lax.axis_index("c")
    s = lax.axis_index("s")
    pltpu.sync_copy(oblk, ones)
    pltpu.sync_copy(zblk, acc.at[pl.ds(s * RPT, RPT)])
    plsc.subcore_barrier()

    def chunk(ss, carry):
        r0 = c * ER + s * TR + ss * CR
        pltpu.sync_copy(dsts.at[pl.ds(r0, CR)], didx)
        for r in range(CR):
            pltpu.async_copy(ones, acc.at[didx.at[r]], sem, add=True)
            if r >= LAG:
                pltpu.make_async_copy(ones, acc.at[didx.at[r - LAG]],
                                      sem).wait()
        for r in range(CR - LAG, CR):
            pltpu.make_async_copy(ones, acc.at[didx.at[r]], sem).wait()
        return carry

    lax.fori_loop(0, NSUP, chunk, 0)
    plsc.subcore_barrier()
    pltpu.sync_copy(acc.at[pl.ds(s * RPT, RPT)],
                    out.at[pl.ds(c * NACC + s * RPT, RPT)])


R = 2000
G = N // R


def _safe(h):
    h = jnp.nan_to_num(h, nan=0.0, posinf=1.0, neginf=-1.0)
    return jnp.clip(h, -10.0, 10.0)


def _leaky(h):
    return jnp.where(h >= 0, h, 0.1 * h)


def _proj_body(x_ref, w_ref, b_ref, o_ref):
    x = jnp.nan_to_num(x_ref[...])
    h = jnp.dot(x, w_ref[...], preferred_element_type=jnp.float32) + b_ref[...]
    h = _safe(h)
    o_ref[0] = h[:, :32]
    o_ref[1] = h[:, 32:]


def _proj(x, w, b):
    return pl.pallas_call(
        _proj_body,
        grid=(G,),
        in_specs=[
            pl.BlockSpec((R, DIN), lambda i: (i, 0)),
            pl.BlockSpec((DIN, DH), lambda i: (0, 0)),
            pl.BlockSpec((1, DH), lambda i: (0, 0)),
        ],
        out_specs=pl.BlockSpec((2, R, 32), lambda i: (0, i, 0)),
        out_shape=jax.ShapeDtypeStruct((2, N, 32), jnp.float32),
    )(x, w, b.reshape(1, DH))


def _combine_core(s_ref, c_ref, h_ref, w_ref, b_ref):
    cnt = c_ref[...]
    inv = 1.0 / jnp.maximum(cnt[:, 0:1], 1.0)
    mean = jnp.concatenate([s_ref[0], s_ref[1]], axis=1) * inv
    xd = jnp.concatenate([h_ref[0], h_ref[1]], axis=1)
    z = jnp.concatenate([mean, xd], axis=1)
    h = jnp.dot(z, w_ref[...], preferred_element_type=jnp.float32) + b_ref[...]
    return _leaky(_safe(h))


def _combine1_body(s_ref, c_ref, h_ref, w_ref, b_ref, o_ref):
    h = _combine_core(s_ref, c_ref, h_ref, w_ref, b_ref)
    o_ref[0] = h[:, :32]
    o_ref[1] = h[:, 32:]


def _combine1(S, cnt, h, wful, bl):
    return pl.pallas_call(
        _combine1_body,
        grid=(G,),
        in_specs=[
            pl.BlockSpec((2, R, 32), lambda i: (0, i, 0)),
            pl.BlockSpec((R, CW), lambda i: (i, 0)),
            pl.BlockSpec((2, R, 32), lambda i: (0, i, 0)),
            pl.BlockSpec((2 * DH, DH), lambda i: (0, 0)),
            pl.BlockSpec((1, DH), lambda i: (0, 0)),
        ],
        out_specs=pl.BlockSpec((2, R, 32), lambda i: (0, i, 0)),
        out_shape=jax.ShapeDtypeStruct((2, N, 32), jnp.float32),
    )(S, cnt, h, wful, bl.reshape(1, DH))


def _combine2_body(s_ref, c_ref, h_ref, w_ref, b_ref, o_ref):
    o_ref[...] = _combine_core(s_ref, c_ref, h_ref, w_ref, b_ref)


def _combine2(S, cnt, h, wful, bl):
    return pl.pallas_call(
        _combine2_body,
        grid=(G,),
        in_specs=[
            pl.BlockSpec((2, R, 32), lambda i: (0, i, 0)),
            pl.BlockSpec((R, CW), lambda i: (i, 0)),
            pl.BlockSpec((2, R, 32), lambda i: (0, i, 0)),
            pl.BlockSpec((2 * DH, DO), lambda i: (0, 0)),
            pl.BlockSpec((1, DO), lambda i: (0, 0)),
        ],
        out_specs=pl.BlockSpec((R, DO), lambda i: (i, 0)),
        out_shape=jax.ShapeDtypeStruct((N, DO), jnp.float32),
    )(S, cnt, h, wful, bl.reshape(1, DO))


def _user_body(s_ref, c_ref, h_ref, w_ref, b_ref, w1_ref, b1_ref, w2_ref,
               b2_ref, o_ref, p_ref):
    h = _combine_core(s_ref, c_ref, h_ref, w_ref, b_ref)
    o_ref[...] = h
    z = _leaky(jnp.dot(h, w1_ref[...], preferred_element_type=jnp.float32)
               + b1_ref[...])
    p = jnp.dot(z, w2_ref[...], preferred_element_type=jnp.float32) + b2_ref[...]
    p_ref[...] = jax.nn.sigmoid(p)


def _combine2_user(S, cnt, h, wful, bl, w1, b1, w2, b2):
    return pl.pallas_call(
        _user_body,
        grid=(G,),
        in_specs=[
            pl.BlockSpec((2, R, 32), lambda i: (0, i, 0)),
            pl.BlockSpec((R, CW), lambda i: (i, 0)),
            pl.BlockSpec((2, R, 32), lambda i: (0, i, 0)),
            pl.BlockSpec((2 * DH, DO), lambda i: (0, 0)),
            pl.BlockSpec((1, DO), lambda i: (0, 0)),
            pl.BlockSpec((DO, 16), lambda i: (0, 0)),
            pl.BlockSpec((1, 16), lambda i: (0, 0)),
            pl.BlockSpec((16, 1), lambda i: (0, 0)),
            pl.BlockSpec((1, 1), lambda i: (0, 0)),
        ],
        out_specs=[
            pl.BlockSpec((R, DO), lambda i: (i, 0)),
            pl.BlockSpec((R, 1), lambda i: (i, 0)),
        ],
        out_shape=[
            jax.ShapeDtypeStruct((N, DO), jnp.float32),
            jax.ShapeDtypeStruct((N, 1), jnp.float32),
        ],
    )(S, cnt, h, wful, bl.reshape(1, DO), w1, b1.reshape(1, 16), w2,
      b2.reshape(1, 1))


def _edge_arrays(edge_index):
    ei = edge_index.astype(jnp.int32)
    src = jnp.concatenate([ei[0], jnp.zeros((PAD,), jnp.int32)])
    dst = jnp.concatenate([ei[1], jnp.full((PAD,), N, jnp.int32)])
    src2 = jnp.concatenate([src, src + N]).reshape(2 * ER, 128)
    return src2, dst.reshape(ER, 128), dst


def kernel(x_user, x_item, edge_index_ui, edge_index_iu,
           proj_user_W, proj_user_b, proj_item_W, proj_item_b,
           conv1_ui_Wl, conv1_ui_bl, conv1_ui_Wr,
           conv1_iu_Wl, conv1_iu_bl, conv1_iu_Wr,
           conv2_ui_Wl, conv2_ui_bl, conv2_ui_Wr,
           conv2_iu_Wl, conv2_iu_bl, conv2_iu_Wr,
           pred_W1, pred_b1, pred_W2, pred_b2):
    src2_ui, dst2_ui, dstp_ui = _edge_arrays(edge_index_ui)
    src2_iu, dst2_iu, dstp_iu = _edge_arrays(edge_index_iu)
    dst_all = jnp.concatenate([dstp_ui, dstp_iu]).reshape(2 * ER, 128)
    zblk = jnp.zeros((RPT, 32), jnp.float32)
    zblkc = jnp.zeros((RPT, CW), jnp.float32)
    oblk = jnp.ones((128, CW), jnp.float32)

    counts = _sc_counts(dst_all, oblk, zblkc).reshape(2, NACC, CW)
    cnt_item = counts[0]
    cnt_user = counts[1]

    hu = _proj(x_user, proj_user_W, proj_user_b)
    hi = _proj(x_item, proj_item_W, proj_item_b)

    s_ui = _sc_segsum(hu.reshape(2 * N, 32), src2_ui, dst2_ui, zblk)
    s_iu = _sc_segsum(hi.reshape(2 * N, 32), src2_iu, dst2_iu, zblk)

    w1_ui = jnp.concatenate([conv1_ui_Wl, conv1_ui_Wr], axis=0)
    w1_iu = jnp.concatenate([conv1_iu_Wl, conv1_iu_Wr], axis=0)
    hi1 = _combine1(s_ui.reshape(2, NACC, 32), cnt_item, hi, w1_ui, conv1_ui_bl)
    hu1 = _combine1(s_iu.reshape(2, NACC, 32), cnt_user, hu, w1_iu, conv1_iu_bl)

    s2_ui = _sc_segsum(hu1.reshape(2 * N, 32), src2_ui, dst2_ui, zblk)
    s2_iu = _sc_segsum(hi1.reshape(2 * N, 32), src2_iu, dst2_iu, zblk)

    w2_ui = jnp.concatenate([conv2_ui_Wl, conv2_ui_Wr], axis=0)
    w2_iu = jnp.concatenate([conv2_iu_Wl, conv2_iu_Wr], axis=0)
    hi2 = _combine2(s2_ui.reshape(2, NACC, 32), cnt_item, hi1, w2_ui,
                    conv2_ui_bl)
    hu2, pred = _combine2_user(s2_iu.reshape(2, NACC, 32), cnt_user, hu1,
                               w2_iu, conv2_iu_bl, pred_W1, pred_b1,
                               pred_W2, pred_b2)
    return pred[:, 0], {"user": hu2, "item": hi2}


# speedup vs baseline: 5.2954x; 1.0061x over previous
import functools

import jax
import jax.numpy as jnp
from jax import lax
from jax.experimental import pallas as pl
from jax.experimental.pallas import tpu as pltpu
from jax.experimental.pallas import tpu_sc as plsc

N = 50000
E = 800000
DIN = 128
DH = 64
DO = 32
NC = 2
NS = 16
EP = 819200
PAD = EP - E
NACC = 50176
RPT = NACC // NS
K = 128
NBLK = EP // (NS * K)
ER = EP // 128
TR = NBLK
SUP = 40
NSUP = NBLK // SUP
CR = SUP
NBUF = 4
AHEAD = NBUF - 1

_MESH = plsc.VectorSubcoreMesh(
    core_axis_name="c", subcore_axis_name="s", num_cores=NC, num_subcores=NS)


@functools.partial(
    pl.kernel,
    out_type=jax.ShapeDtypeStruct((NC * NACC, 32), jnp.float32),
    mesh=_MESH,
    scratch_types=[
        pltpu.VMEM((CR, 128), jnp.int32),
        pltpu.VMEM((CR, 128), jnp.int32),
        pltpu.VMEM((NBUF, K, 32), jnp.float32),
        pltpu.VMEM_SHARED((NACC, 32), jnp.float32),
        [pltpu.SemaphoreType.DMA] * NBUF,
        [pltpu.SemaphoreType.DMA] * NBUF,
    ],
    compiler_params=pltpu.CompilerParams(use_tc_tiling_on_sc=False),
)
def _sc_segsum(table, src2, dst2, zblk, out, sidx, didx, rows, acc,
               gsem, ssem):
    c = lax.axis_index("c")
    s = lax.axis_index("s")
    pltpu.sync_copy(zblk, acc.at[pl.ds(s * RPT, RPT)])
    plsc.subcore_barrier()

    def gfire(sb):
        p = sb % NBUF
        pltpu.async_copy(table.at[sidx.at[sb]], rows.at[p], gsem[p])

    def gdrain(sb):
        p = sb % NBUF
        pltpu.make_async_copy(table.at[sidx.at[sb]], rows.at[p],
                              gsem[p]).wait()

    def sfire(sb):
        p = sb % NBUF
        pltpu.async_copy(rows.at[p], acc.at[didx.at[sb]], ssem[p], add=True)

    def sdrain(sb):
        p = sb % NBUF
        pltpu.make_async_copy(rows.at[p], acc.at[didx.at[sb]],
                              ssem[p]).wait()

    def chunk(ss, carry):
        r0 = s * TR + ss * CR
        pltpu.sync_copy(src2.at[pl.ds(c * ER + r0, CR)], sidx)
        pltpu.sync_copy(dst2.at[pl.ds(r0, CR)], didx)
        for a in range(AHEAD):
            gfire(a)
        for sb in range(SUP):
            nb = sb + AHEAD
            if nb < SUP:
                if sb >= 1:
                    sdrain(sb - 1)
                gfire(nb)
            gdrain(sb)
            sfire(sb)
        for b in range(SUP - NBUF, SUP):
            sdrain(b)
        return carry

    lax.fori_loop(0, NSUP, chunk, 0)
    plsc.subcore_barrier()
    pltpu.sync_copy(acc.at[pl.ds(s * RPT, RPT)],
                    out.at[pl.ds(c * NACC + s * RPT, RPT)])


CW = 16
LAG = 4


@functools.partial(
    pl.kernel,
    out_type=jax.ShapeDtypeStruct((NC * NACC, CW), jnp.float32),
    mesh=_MESH,
    scratch_types=[
        pltpu.VMEM((CR, 128), jnp.int32),
        pltpu.VMEM((128, CW), jnp.float32),
        pltpu.VMEM_SHARED((NACC, CW), jnp.float32),
        pltpu.SemaphoreType.DMA,
    ],
    compiler_params=pltpu.CompilerParams(use_tc_tiling_on_sc=False),
)
def _sc_counts(dsts, oblk, zblk, out, didx, ones, acc, sem):
    c = lax.axis_index("c")
    s = lax.axis_index("s")
    pltpu.sync_copy(oblk, ones)
    pltpu.sync_copy(zblk, acc.at[pl.ds(s * RPT, RPT)])
    plsc.subcore_barrier()

    def chunk(ss, carry):
        r0 = c * ER + s * TR + ss * CR
        pltpu.sync_copy(dsts.at[pl.ds(r0, CR)], didx)
        for r in range(CR):
            pltpu.async_copy(ones, acc.at[didx.at[r]], sem, add=True)
            if r >= LAG:
                pltpu.make_async_copy(ones, acc.at[didx.at[r - LAG]],
                                      sem).wait()
        for r in range(CR - LAG, CR):
            pltpu.make_async_copy(ones, acc.at[didx.at[r]], sem).wait()
        return carry

    lax.fori_loop(0, NSUP, chunk, 0)
    plsc.subcore_barrier()
    pltpu.sync_copy(acc.at[pl.ds(s * RPT, RPT)],
                    out.at[pl.ds(c * NACC + s * RPT, RPT)])


R = 2000
G = N // R


def _safe(h):
    h = jnp.nan_to_num(h, nan=0.0, posinf=1.0, neginf=-1.0)
    return jnp.clip(h, -10.0, 10.0)


def _leaky(h):
    return jnp.where(h >= 0, h, 0.1 * h)


def _proj_body(x_ref, w_ref, b_ref, o_ref):
    x = jnp.nan_to_num(x_ref[...])
    h = jnp.dot(x, w_ref[...], preferred_element_type=jnp.float32) + b_ref[...]
    h = _safe(h)
    o_ref[0] = h[:, :32]
    o_ref[1] = h[:, 32:]


def _proj(x, w, b):
    return pl.pallas_call(
        _proj_body,
        grid=(G,),
        in_specs=[
            pl.BlockSpec((R, DIN), lambda i: (i, 0)),
            pl.BlockSpec((DIN, DH), lambda i: (0, 0)),
            pl.BlockSpec((1, DH), lambda i: (0, 0)),
        ],
        out_specs=pl.BlockSpec((2, R, 32), lambda i: (0, i, 0)),
        out_shape=jax.ShapeDtypeStruct((2, N, 32), jnp.float32),
    )(x, w, b.reshape(1, DH))


def _combine_core(s_ref, c_ref, h_ref, w_ref, b_ref):
    cnt = c_ref[...]
    inv = 1.0 / jnp.maximum(cnt[:, 0:1], 1.0)
    mean = jnp.concatenate([s_ref[0], s_ref[1]], axis=1) * inv
    xd = jnp.concatenate([h_ref[0], h_ref[1]], axis=1)
    z = jnp.concatenate([mean, xd], axis=1)
    h = jnp.dot(z, w_ref[...], preferred_element_type=jnp.float32) + b_ref[...]
    return _leaky(_safe(h))


def _combine1_body(s_ref, c_ref, h_ref, w_ref, b_ref, o_ref):
    h = _combine_core(s_ref, c_ref, h_ref, w_ref, b_ref)
    o_ref[0] = h[:, :32]
    o_ref[1] = h[:, 32:]


def _combine1(S, cnt, h, wful, bl):
    return pl.pallas_call(
        _combine1_body,
        grid=(G,),
        in_specs=[
            pl.BlockSpec((2, R, 32), lambda i: (0, i, 0)),
            pl.BlockSpec((R, CW), lambda i: (i, 0)),
            pl.BlockSpec((2, R, 32), lambda i: (0, i, 0)),
            pl.BlockSpec((2 * DH, DH), lambda i: (0, 0)),
            pl.BlockSpec((1, DH), lambda i: (0, 0)),
        ],
        out_specs=pl.BlockSpec((2, R, 32), lambda i: (0, i, 0)),
        out_shape=jax.ShapeDtypeStruct((2, N, 32), jnp.float32),
    )(S, cnt, h, wful, bl.reshape(1, DH))


def _combine2_body(s_ref, c_ref, h_ref, w_ref, b_ref, o_ref):
    o_ref[...] = _combine_core(s_ref, c_ref, h_ref, w_ref, b_ref)


def _combine2(S, cnt, h, wful, bl):
    return pl.pallas_call(
        _combine2_body,
        grid=(G,),
        in_specs=[
            pl.BlockSpec((2, R, 32), lambda i: (0, i, 0)),
            pl.BlockSpec((R, CW), lambda i: (i, 0)),
            pl.BlockSpec((2, R, 32), lambda i: (0, i, 0)),
            pl.BlockSpec((2 * DH, DO), lambda i: (0, 0)),
            pl.BlockSpec((1, DO), lambda i: (0, 0)),
        ],
        out_specs=pl.BlockSpec((R, DO), lambda i: (i, 0)),
        out_shape=jax.ShapeDtypeStruct((N, DO), jnp.float32),
    )(S, cnt, h, wful, bl.reshape(1, DO))


def _user_body(s_ref, c_ref, h_ref, w_ref, b_ref, w1_ref, b1_ref, w2_ref,
               b2_ref, o_ref, p_ref):
    h = _combine_core(s_ref, c_ref, h_ref, w_ref, b_ref)
    o_ref[...] = h
    z = _leaky(jnp.dot(h, w1_ref[...], preferred_element_type=jnp.float32)
               + b1_ref[...])
    p = jnp.dot(z, w2_ref[...], preferred_element_type=jnp.float32) + b2_ref[...]
    p_ref[...] = jax.nn.sigmoid(p)


def _combine2_user(S, cnt, h, wful, bl, w1, b1, w2, b2):
    return pl.pallas_call(
        _user_body,
        grid=(G,),
        in_specs=[
            pl.BlockSpec((2, R, 32), lambda i: (0, i, 0)),
            pl.BlockSpec((R, CW), lambda i: (i, 0)),
            pl.BlockSpec((2, R, 32), lambda i: (0, i, 0)),
            pl.BlockSpec((2 * DH, DO), lambda i: (0, 0)),
            pl.BlockSpec((1, DO), lambda i: (0, 0)),
            pl.BlockSpec((DO, 16), lambda i: (0, 0)),
            pl.BlockSpec((1, 16), lambda i: (0, 0)),
            pl.BlockSpec((16, 1), lambda i: (0, 0)),
            pl.BlockSpec((1, 1), lambda i: (0, 0)),
        ],
        out_specs=[
            pl.BlockSpec((R, DO), lambda i: (i, 0)),
            pl.BlockSpec((R, 1), lambda i: (i, 0)),
        ],
        out_shape=[
            jax.ShapeDtypeStruct((N, DO), jnp.float32),
            jax.ShapeDtypeStruct((N, 1), jnp.float32),
        ],
    )(S, cnt, h, wful, bl.reshape(1, DO), w1, b1.reshape(1, 16), w2,
      b2.reshape(1, 1))


def _edge_arrays(edge_index):
    ei = edge_index.astype(jnp.int32)
    src = jnp.concatenate([ei[0], jnp.zeros((PAD,), jnp.int32)])
    dst = jnp.concatenate([ei[1], jnp.full((PAD,), N, jnp.int32)])
    src2 = jnp.concatenate([src, src + N]).reshape(2 * ER, 128)
    return src2, dst.reshape(ER, 128), dst


def kernel(x_user, x_item, edge_index_ui, edge_index_iu,
           proj_user_W, proj_user_b, proj_item_W, proj_item_b,
           conv1_ui_Wl, conv1_ui_bl, conv1_ui_Wr,
           conv1_iu_Wl, conv1_iu_bl, conv1_iu_Wr,
           conv2_ui_Wl, conv2_ui_bl, conv2_ui_Wr,
           conv2_iu_Wl, conv2_iu_bl, conv2_iu_Wr,
           pred_W1, pred_b1, pred_W2, pred_b2):
    src2_ui, dst2_ui, dstp_ui = _edge_arrays(edge_index_ui)
    src2_iu, dst2_iu, dstp_iu = _edge_arrays(edge_index_iu)
    dst_all = jnp.concatenate([dstp_ui, dstp_iu]).reshape(2 * ER, 128)
    zblk = jnp.zeros((RPT, 32), jnp.float32)
    zblkc = jnp.zeros((RPT, CW), jnp.float32)
    oblk = jnp.ones((128, CW), jnp.float32)

    counts = _sc_counts(dst_all, oblk, zblkc).reshape(2, NACC, CW)
    cnt_item = counts[0]
    cnt_user = counts[1]

    hu = _proj(x_user, proj_user_W, proj_user_b)
    hi = _proj(x_item, proj_item_W, proj_item_b)

    s_ui = _sc_segsum(hu.reshape(2 * N, 32), src2_ui, dst2_ui, zblk)
    s_iu = _sc_segsum(hi.reshape(2 * N, 32), src2_iu, dst2_iu, zblk)

    w1_ui = jnp.concatenate([conv1_ui_Wl, conv1_ui_Wr], axis=0)
    w1_iu = jnp.concatenate([conv1_iu_Wl, conv1_iu_Wr], axis=0)
    hi1 = _combine1(s_ui.reshape(2, NACC, 32), cnt_item, hi, w1_ui, conv1_ui_bl)
    hu1 = _combine1(s_iu.reshape(2, NACC, 32), cnt_user, hu, w1_iu, conv1_iu_bl)

    s2_ui = _sc_segsum(hu1.reshape(2 * N, 32), src2_ui, dst2_ui, zblk)
    s2_iu = _sc_segsum(hi1.reshape(2 * N, 32), src2_iu, dst2_iu, zblk)

    w2_ui = jnp.concatenate([conv2_ui_Wl, conv2_ui_Wr], axis=0)
    w2_iu = jnp.concatenate([conv2_iu_Wl, conv2_iu_Wr], axis=0)
    hi2 = _combine2(s2_ui.reshape(2, NACC, 32), cnt_item, hi1, w2_ui,
                    conv2_ui_bl)
    hu2, pred = _combine2_user(s2_iu.reshape(2, NACC, 32), cnt_user, hu1,
                               w2_iu, conv2_iu_bl, pred_W1, pred_b1,
                               pred_W2, pred_b2)
    return pred[:, 0], {"user": hu2, "item": hi2}


# speedup vs baseline: 6.0724x; 1.1467x over previous
import functools

import jax
import jax.numpy as jnp
from jax import lax
from jax.experimental import pallas as pl
from jax.experimental.pallas import tpu as pltpu
from jax.experimental.pallas import tpu_sc as plsc

N = 50000
E = 800000
DIN = 128
DH = 64
DO = 32
NC = 2
NS = 16
EP = 819200
PAD = EP - E
NACC = 50176
RPT = NACC // NS
K = 128
NBLK = EP // (NS * K)
ER = EP // 128
TR = NBLK
SUP = 40
NSUP = NBLK // SUP
CR = SUP
NBUF = 4
AHEAD = NBUF - 1
T2R = EP // (128 * NC * NS)
NSUP2 = T2R // CR

_MESH = plsc.VectorSubcoreMesh(
    core_axis_name="c", subcore_axis_name="s", num_cores=NC, num_subcores=NS)


@functools.partial(
    pl.kernel,
    out_type=jax.ShapeDtypeStruct((NC * NACC, 32), jnp.float32),
    mesh=_MESH,
    scratch_types=[
        pltpu.VMEM((CR, 128), jnp.int32),
        pltpu.VMEM((CR, 128), jnp.int32),
        pltpu.VMEM((NBUF, K, 32), jnp.float32),
        pltpu.VMEM_SHARED((NACC, 32), jnp.float32),
        [pltpu.SemaphoreType.DMA] * NBUF,
        [pltpu.SemaphoreType.DMA] * NBUF,
    ],
    compiler_params=pltpu.CompilerParams(use_tc_tiling_on_sc=False),
)
def _sc_segsum(table, src2, dst2, zblk, out, sidx, didx, rows, acc,
               gsem, ssem):
    c = lax.axis_index("c")
    s = lax.axis_index("s")
    pltpu.sync_copy(zblk, acc.at[pl.ds(s * RPT, RPT)])
    plsc.subcore_barrier()

    _seg_pipeline(table, src2, dst2, sidx, didx, rows, acc, gsem, ssem,
                  NSUP, c * ER + s * TR, s * TR)
    plsc.subcore_barrier()
    pltpu.sync_copy(acc.at[pl.ds(s * RPT, RPT)],
                    out.at[pl.ds(c * NACC + s * RPT, RPT)])


def _seg_pipeline(table, srcr, dstr, sidx, didx, rows, acc, gsem, ssem,
                  nsup, src_base, dst_base):
    def gfire(sb):
        p = sb % NBUF
        pltpu.async_copy(table.at[sidx.at[sb]], rows.at[p], gsem[p])

    def gdrain(sb):
        p = sb % NBUF
        pltpu.make_async_copy(table.at[sidx.at[sb]], rows.at[p],
                              gsem[p]).wait()

    def sfire(sb):
        p = sb % NBUF
        pltpu.async_copy(rows.at[p], acc.at[didx.at[sb]], ssem[p], add=True)

    def sdrain(sb):
        p = sb % NBUF
        pltpu.make_async_copy(rows.at[p], acc.at[didx.at[sb]],
                              ssem[p]).wait()

    def chunk(ss, carry):
        pltpu.sync_copy(srcr.at[pl.ds(src_base + ss * CR, CR)], sidx)
        pltpu.sync_copy(dstr.at[pl.ds(dst_base + ss * CR, CR)], didx)
        for a in range(AHEAD):
            gfire(a)
        for sb in range(SUP):
            nb = sb + AHEAD
            if nb < SUP:
                if sb >= 1:
                    sdrain(sb - 1)
                gfire(nb)
            gdrain(sb)
            sfire(sb)
        for b in range(SUP - NBUF, SUP):
            sdrain(b)
        return carry

    lax.fori_loop(0, nsup, chunk, 0)


@functools.partial(
    pl.kernel,
    out_type=jax.ShapeDtypeStruct((NC * NACC, 32), jnp.float32),
    mesh=_MESH,
    scratch_types=[
        pltpu.VMEM((CR, 128), jnp.int32),
        pltpu.VMEM((CR, 128), jnp.int32),
        pltpu.VMEM((NBUF, K, 32), jnp.float32),
        pltpu.VMEM_SHARED((NACC, 32), jnp.float32),
        [pltpu.SemaphoreType.DMA] * NBUF,
        [pltpu.SemaphoreType.DMA] * NBUF,
    ],
    compiler_params=pltpu.CompilerParams(use_tc_tiling_on_sc=False),
)
def _sc_segsum_e(table, srcr, dstr, zblk, out, sidx, didx, rows, acc,
                 gsem, ssem):
    c = lax.axis_index("c")
    s = lax.axis_index("s")
    pltpu.sync_copy(zblk, acc.at[pl.ds(s * RPT, RPT)])
    plsc.subcore_barrier()
    base = (c * NS + s) * T2R
    _seg_pipeline(table, srcr, dstr, sidx, didx, rows, acc, gsem, ssem,
                  NSUP2, base, base)
    plsc.subcore_barrier()
    pltpu.sync_copy(acc.at[pl.ds(s * RPT, RPT)],
                    out.at[pl.ds(c * NACC + s * RPT, RPT)])


CW = 16
LAG = 4


@functools.partial(
    pl.kernel,
    out_type=jax.ShapeDtypeStruct((NC * NACC, CW), jnp.float32),
    mesh=_MESH,
    scratch_types=[
        pltpu.VMEM((CR, 128), jnp.int32),
        pltpu.VMEM((128, CW), jnp.float32),
        pltpu.VMEM_SHARED((NACC, CW), jnp.float32),
        pltpu.SemaphoreType.DMA,
    ],
    compiler_params=pltpu.CompilerParams(use_tc_tiling_on_sc=False),
)
def _sc_counts(dsts, oblk, zblk, out, didx, ones, acc, sem):
    c = lax.axis_index("c")
    s = lax.axis_index("s")
    pltpu.sync_copy(oblk, ones)
    pltpu.sync_copy(zblk, acc.at[pl.ds(s * RPT, RPT)])
    plsc.subcore_barrier()

    def chunk(ss, carry):
        r0 = c * ER + s * TR + ss * CR
        pltpu.sync_copy(dsts.at[pl.ds(r0, CR)], didx)
        for r in range(CR):
            pltpu.async_copy(ones, acc.at[didx.at[r]], sem, add=True)
            if r >= LAG:
                pltpu.make_async_copy(ones, acc.at[didx.at[r - LAG]],
                                      sem).wait()
        for r in range(CR - LAG, CR):
            pltpu.make_async_copy(ones, acc.at[didx.at[r]], sem).wait()
        return carry

    lax.fori_loop(0, NSUP, chunk, 0)
    plsc.subcore_barrier()
    pltpu.sync_copy(acc.at[pl.ds(s * RPT, RPT)],
                    out.at[pl.ds(c * NACC + s * RPT, RPT)])


R = 2000
G = N // R


def _safe(h):
    h = jnp.nan_to_num(h, nan=0.0, posinf=1.0, neginf=-1.0)
    return jnp.clip(h, -10.0, 10.0)


def _leaky(h):
    return jnp.where(h >= 0, h, 0.1 * h)


def _proj_body(x_ref, w_ref, b_ref, o_ref):
    x = jnp.nan_to_num(x_ref[...])
    h = jnp.dot(x, w_ref[...], preferred_element_type=jnp.float32) + b_ref[...]
    h = _safe(h)
    o_ref[0] = h[:, :32]
    o_ref[1] = h[:, 32:]


def _proj(x, w, b):
    return pl.pallas_call(
        _proj_body,
        grid=(G,),
        in_specs=[
            pl.BlockSpec((R, DIN), lambda i: (i, 0)),
            pl.BlockSpec((DIN, DH), lambda i: (0, 0)),
            pl.BlockSpec((1, DH), lambda i: (0, 0)),
        ],
        out_specs=pl.BlockSpec((2, R, 32), lambda i: (0, i, 0)),
        out_shape=jax.ShapeDtypeStruct((2, N, 32), jnp.float32),
    )(x, w, b.reshape(1, DH))


def _combine_core(s_ref, c_ref, h_ref, w_ref, b_ref):
    cnt = c_ref[...]
    inv = 1.0 / jnp.maximum(cnt[:, 0:1], 1.0)
    mean = jnp.concatenate([s_ref[0], s_ref[1]], axis=1) * inv
    xd = jnp.concatenate([h_ref[0], h_ref[1]], axis=1)
    z = jnp.concatenate([mean, xd], axis=1)
    h = jnp.dot(z, w_ref[...], preferred_element_type=jnp.float32) + b_ref[...]
    return _leaky(_safe(h))


def _combine1_body(s_ref, c_ref, h_ref, w_ref, b_ref, wl2_ref, wr2_ref,
                   t_ref, x_ref):
    h = _combine_core(s_ref, c_ref, h_ref, w_ref, b_ref)
    t_ref[...] = jnp.dot(h, wl2_ref[...], preferred_element_type=jnp.float32)
    x_ref[...] = jnp.dot(h, wr2_ref[...], preferred_element_type=jnp.float32)


def _combine1(S, cnt, h, wful, bl, wl2, wr2):
    return pl.pallas_call(
        _combine1_body,
        grid=(G,),
        in_specs=[
            pl.BlockSpec((2, R, 32), lambda i: (0, i, 0)),
            pl.BlockSpec((R, CW), lambda i: (i, 0)),
            pl.BlockSpec((2, R, 32), lambda i: (0, i, 0)),
            pl.BlockSpec((2 * DH, DH), lambda i: (0, 0)),
            pl.BlockSpec((1, DH), lambda i: (0, 0)),
            pl.BlockSpec((DH, DO), lambda i: (0, 0)),
            pl.BlockSpec((DH, DO), lambda i: (0, 0)),
        ],
        out_specs=[
            pl.BlockSpec((R, DO), lambda i: (i, 0)),
            pl.BlockSpec((R, DO), lambda i: (i, 0)),
        ],
        out_shape=[
            jax.ShapeDtypeStruct((N, DO), jnp.float32),
            jax.ShapeDtypeStruct((N, DO), jnp.float32),
        ],
    )(S, cnt, h, wful, bl.reshape(1, DH), wl2, wr2)


def _layer2_core(s_ref, c_ref, x_ref, b_ref):
    cnt = c_ref[...]
    inv = 1.0 / jnp.maximum(cnt[:, 0:1], 1.0)
    h = (s_ref[0] + s_ref[1]) * inv + x_ref[...] + b_ref[...]
    return _leaky(_safe(h))


def _combine2_body(s_ref, c_ref, x_ref, b_ref, o_ref):
    o_ref[...] = _layer2_core(s_ref, c_ref, x_ref, b_ref)


def _combine2(S, cnt, xr, bl):
    return pl.pallas_call(
        _combine2_body,
        grid=(G,),
        in_specs=[
            pl.BlockSpec((2, R, 32), lambda i: (0, i, 0)),
            pl.BlockSpec((R, CW), lambda i: (i, 0)),
            pl.BlockSpec((R, DO), lambda i: (i, 0)),
            pl.BlockSpec((1, DO), lambda i: (0, 0)),
        ],
        out_specs=pl.BlockSpec((R, DO), lambda i: (i, 0)),
        out_shape=jax.ShapeDtypeStruct((N, DO), jnp.float32),
    )(S, cnt, xr, bl.reshape(1, DO))


def _user_body(s_ref, c_ref, x_ref, b_ref, w1_ref, b1_ref, w2_ref,
               b2_ref, o_ref, p_ref):
    h = _layer2_core(s_ref, c_ref, x_ref, b_ref)
    o_ref[...] = h
    z = _leaky(jnp.dot(h, w1_ref[...], preferred_element_type=jnp.float32)
               + b1_ref[...])
    p = jnp.dot(z, w2_ref[...], preferred_element_type=jnp.float32) + b2_ref[...]
    p_ref[...] = jax.nn.sigmoid(p)


def _combine2_user(S, cnt, xr, bl, w1, b1, w2, b2):
    return pl.pallas_call(
        _user_body,
        grid=(G,),
        in_specs=[
            pl.BlockSpec((2, R, 32), lambda i: (0, i, 0)),
            pl.BlockSpec((R, CW), lambda i: (i, 0)),
            pl.BlockSpec((R, DO), lambda i: (i, 0)),
            pl.BlockSpec((1, DO), lambda i: (0, 0)),
            pl.BlockSpec((DO, 16), lambda i: (0, 0)),
            pl.BlockSpec((1, 16), lambda i: (0, 0)),
            pl.BlockSpec((16, 1), lambda i: (0, 0)),
            pl.BlockSpec((1, 1), lambda i: (0, 0)),
        ],
        out_specs=[
            pl.BlockSpec((R, DO), lambda i: (i, 0)),
            pl.BlockSpec((R, 1), lambda i: (i, 0)),
        ],
        out_shape=[
            jax.ShapeDtypeStruct((N, DO), jnp.float32),
            jax.ShapeDtypeStruct((N, 1), jnp.float32),
        ],
    )(S, cnt, xr, bl.reshape(1, DO), w1, b1.reshape(1, 16), w2,
      b2.reshape(1, 1))


def _edge_arrays(edge_index):
    ei = edge_index.astype(jnp.int32)
    src = jnp.concatenate([ei[0], jnp.zeros((PAD,), jnp.int32)])
    dst = jnp.concatenate([ei[1], jnp.full((PAD,), N, jnp.int32)])
    src2 = jnp.concatenate([src, src + N]).reshape(2 * ER, 128)
    return src2, src.reshape(ER, 128), dst.reshape(ER, 128), dst


def kernel(x_user, x_item, edge_index_ui, edge_index_iu,
           proj_user_W, proj_user_b, proj_item_W, proj_item_b,
           conv1_ui_Wl, conv1_ui_bl, conv1_ui_Wr,
           conv1_iu_Wl, conv1_iu_bl, conv1_iu_Wr,
           conv2_ui_Wl, conv2_ui_bl, conv2_ui_Wr,
           conv2_iu_Wl, conv2_iu_bl, conv2_iu_Wr,
           pred_W1, pred_b1, pred_W2, pred_b2):
    src2_ui, srcr_ui, dst2_ui, dstp_ui = _edge_arrays(edge_index_ui)
    src2_iu, srcr_iu, dst2_iu, dstp_iu = _edge_arrays(edge_index_iu)
    dst_all = jnp.concatenate([dstp_ui, dstp_iu]).reshape(2 * ER, 128)
    zblk = jnp.zeros((RPT, 32), jnp.float32)
    zblkc = jnp.zeros((RPT, CW), jnp.float32)
    oblk = jnp.ones((128, CW), jnp.float32)

    counts = _sc_counts(dst_all, oblk, zblkc).reshape(2, NACC, CW)
    cnt_item = counts[0]
    cnt_user = counts[1]

    hu = _proj(x_user, proj_user_W, proj_user_b)
    hi = _proj(x_item, proj_item_W, proj_item_b)

    s_ui = _sc_segsum(hu.reshape(2 * N, 32), src2_ui, dst2_ui, zblk)
    s_iu = _sc_segsum(hi.reshape(2 * N, 32), src2_iu, dst2_iu, zblk)

    w1_ui = jnp.concatenate([conv1_ui_Wl, conv1_ui_Wr], axis=0)
    w1_iu = jnp.concatenate([conv1_iu_Wl, conv1_iu_Wr], axis=0)
    t2i, xri = _combine1(s_ui.reshape(2, NACC, 32), cnt_item, hi, w1_ui,
                         conv1_ui_bl, conv2_iu_Wl, conv2_ui_Wr)
    t2u, xru = _combine1(s_iu.reshape(2, NACC, 32), cnt_user, hu, w1_iu,
                         conv1_iu_bl, conv2_ui_Wl, conv2_iu_Wr)

    s2_ui = _sc_segsum_e(t2u, srcr_ui, dst2_ui, zblk).reshape(2, NACC, 32)
    s2_iu = _sc_segsum_e(t2i, srcr_iu, dst2_iu, zblk).reshape(2, NACC, 32)

    hi2 = _combine2(s2_ui, cnt_item, xri, conv2_ui_bl)
    hu2, pred = _combine2_user(s2_iu, cnt_user, xru, conv2_iu_bl,
                               pred_W1, pred_b1, pred_W2, pred_b2)
    return pred[:, 0], {"user": hu2, "item": hi2}


# speedup vs baseline: 11.4695x; 1.8888x over previous
import functools

import jax
import jax.numpy as jnp
from jax import lax
from jax.experimental import pallas as pl
from jax.experimental.pallas import tpu as pltpu
from jax.experimental.pallas import tpu_sc as plsc

N = 50000
E = 800000
DIN = 128
DH = 64
DO = 32
NC = 2
NS = 16
EP = 819200
PAD = EP - E
NACC = 50176
RPT = NACC // NS
K = 128
NBLK = EP // (NS * K)
ER = EP // 128
TR = NBLK
SUP = 40
NSUP = NBLK // SUP
CR = SUP
NBUF = 4
AHEAD = NBUF - 1
T2R = EP // (128 * NC * NS)
NSUP2 = T2R // CR

_MESH = plsc.VectorSubcoreMesh(
    core_axis_name="c", subcore_axis_name="s", num_cores=NC, num_subcores=NS)


@functools.partial(
    pl.kernel,
    out_type=jax.ShapeDtypeStruct((NC * NACC, 32), jnp.float32),
    mesh=_MESH,
    scratch_types=[
        pltpu.VMEM((CR, 128), jnp.int32),
        pltpu.VMEM((CR, 128), jnp.int32),
        pltpu.VMEM((NBUF, K, 32), jnp.float32),
        pltpu.VMEM_SHARED((NACC, 32), jnp.float32),
        [pltpu.SemaphoreType.DMA] * NBUF,
        [pltpu.SemaphoreType.DMA] * NBUF,
    ],
    compiler_params=pltpu.CompilerParams(use_tc_tiling_on_sc=False),
)
def _sc_segsum(table, src2, dst2, zblk, out, sidx, didx, rows, acc,
               gsem, ssem):
    c = lax.axis_index("c")
    s = lax.axis_index("s")
    pltpu.sync_copy(zblk, acc.at[pl.ds(s * RPT, RPT)])
    plsc.subcore_barrier()

    _seg_pipeline(table, src2, dst2, sidx, didx, rows, acc, gsem, ssem,
                  NSUP, c * ER + s * TR, s * TR)
    plsc.subcore_barrier()
    pltpu.sync_copy(acc.at[pl.ds(s * RPT, RPT)],
                    out.at[pl.ds(c * NACC + s * RPT, RPT)])


def _seg_pipeline(table, srcr, dstr, sidx, didx, rows, acc, gsem, ssem,
                  nsup, src_base, dst_base):
    def gfire(sb):
        p = sb % NBUF
        pltpu.async_copy(table.at[sidx.at[sb]], rows.at[p], gsem[p])

    def gdrain(sb):
        p = sb % NBUF
        pltpu.make_async_copy(table.at[sidx.at[sb]], rows.at[p],
                              gsem[p]).wait()

    def sfire(sb):
        p = sb % NBUF
        pltpu.async_copy(rows.at[p], acc.at[didx.at[sb]], ssem[p], add=True)

    def sdrain(sb):
        p = sb % NBUF
        pltpu.make_async_copy(rows.at[p], acc.at[didx.at[sb]],
                              ssem[p]).wait()

    def chunk(ss, carry):
        pltpu.sync_copy(srcr.at[pl.ds(src_base + ss * CR, CR)], sidx)
        pltpu.sync_copy(dstr.at[pl.ds(dst_base + ss * CR, CR)], didx)
        for a in range(AHEAD):
            gfire(a)
        for sb in range(SUP):
            nb = sb + AHEAD
            if nb < SUP:
                if sb >= 1:
                    sdrain(sb - 1)
                gfire(nb)
            gdrain(sb)
            sfire(sb)
        for b in range(SUP - NBUF, SUP):
            sdrain(b)
        return carry

    lax.fori_loop(0, nsup, chunk, 0)


@functools.partial(
    pl.kernel,
    out_type=jax.ShapeDtypeStruct((NC * NACC, 32), jnp.float32),
    mesh=_MESH,
    scratch_types=[
        pltpu.VMEM((CR, 128), jnp.int32),
        pltpu.VMEM((CR, 128), jnp.int32),
        pltpu.VMEM((NBUF, K, 32), jnp.float32),
        pltpu.VMEM_SHARED((NACC, 32), jnp.float32),
        [pltpu.SemaphoreType.DMA] * NBUF,
        [pltpu.SemaphoreType.DMA] * NBUF,
    ],
    compiler_params=pltpu.CompilerParams(use_tc_tiling_on_sc=False),
)
def _sc_segsum_e(table, srcr, dstr, zblk, out, sidx, didx, rows, acc,
                 gsem, ssem):
    c = lax.axis_index("c")
    s = lax.axis_index("s")
    pltpu.sync_copy(zblk, acc.at[pl.ds(s * RPT, RPT)])
    plsc.subcore_barrier()
    base = (c * NS + s) * T2R
    _seg_pipeline(table, srcr, dstr, sidx, didx, rows, acc, gsem, ssem,
                  NSUP2, base, base)
    plsc.subcore_barrier()
    pltpu.sync_copy(acc.at[pl.ds(s * RPT, RPT)],
                    out.at[pl.ds(c * NACC + s * RPT, RPT)])


CW = 16
LAG = 4


@functools.partial(
    pl.kernel,
    out_type=jax.ShapeDtypeStruct((NC * NACC, CW), jnp.float32),
    mesh=_MESH,
    scratch_types=[
        pltpu.VMEM((CR, 128), jnp.int32),
        pltpu.VMEM((128, CW), jnp.float32),
        pltpu.VMEM_SHARED((NACC, CW), jnp.float32),
        pltpu.SemaphoreType.DMA,
    ],
    compiler_params=pltpu.CompilerParams(use_tc_tiling_on_sc=False),
)
def _sc_counts(dsts, oblk, zblk, out, didx, ones, acc, sem):
    c = lax.axis_index("c")
    s = lax.axis_index("s")
    pltpu.sync_copy(oblk, ones)
    pltpu.sync_copy(zblk, acc.at[pl.ds(s * RPT, RPT)])
    plsc.subcore_barrier()

    def chunk(ss, carry):
        r0 = c * ER + s * TR + ss * CR
        pltpu.sync_copy(dsts.at[pl.ds(r0, CR)], didx)
        for r in range(CR):
            pltpu.async_copy(ones, acc.at[didx.at[r]], sem, add=True)
            if r >= LAG:
                pltpu.make_async_copy(ones, acc.at[didx.at[r - LAG]],
                                      sem).wait()
        for r in range(CR - LAG, CR):
            pltpu.make_async_copy(ones, acc.at[didx.at[r]], sem).wait()
        return carry

    lax.fori_loop(0, NSUP, chunk, 0)
    plsc.subcore_barrier()
    pltpu.sync_copy(acc.at[pl.ds(s * RPT, RPT)],
                    out.at[pl.ds(c * NACC + s * RPT, RPT)])


R = 2000
G = N // R


def _safe(h):
    h = jnp.nan_to_num(h, nan=0.0, posinf=1.0, neginf=-1.0)
    return jnp.clip(h, -10.0, 10.0)


def _leaky(h):
    return jnp.where(h >= 0, h, 0.1 * h)


def _proj_body(x_ref, w_ref, b_ref, o_ref):
    x = jnp.nan_to_num(x_ref[...])
    h = jnp.dot(x, w_ref[...], preferred_element_type=jnp.float32) + b_ref[...]
    h = _safe(h)
    o_ref[0] = h[:, :32]
    o_ref[1] = h[:, 32:]


def _proj(x, w, b):
    return pl.pallas_call(
        _proj_body,
        grid=(G,),
        in_specs=[
            pl.BlockSpec((R, DIN), lambda i: (i, 0)),
            pl.BlockSpec((DIN, DH), lambda i: (0, 0)),
            pl.BlockSpec((1, DH), lambda i: (0, 0)),
        ],
        out_specs=pl.BlockSpec((2, R, 32), lambda i: (0, i, 0)),
        out_shape=jax.ShapeDtypeStruct((2, N, 32), jnp.float32),
    )(x, w, b.reshape(1, DH))


def _combine_core(s_ref, c_ref, h_ref, w_ref, b_ref):
    cnt = c_ref[...]
    inv = 1.0 / jnp.maximum(cnt[:, 0:1], 1.0)
    mean = jnp.concatenate([s_ref[0], s_ref[1]], axis=1) * inv
    xd = jnp.concatenate([h_ref[0], h_ref[1]], axis=1)
    z = jnp.concatenate([mean, xd], axis=1)
    h = jnp.dot(z, w_ref[...], preferred_element_type=jnp.float32) + b_ref[...]
    return _leaky(_safe(h))


def _combine1_body(s_ref, c_ref, h_ref, w_ref, b_ref, wl2_ref, wr2_ref,
                   t_ref, x_ref):
    h = _combine_core(s_ref, c_ref, h_ref, w_ref, b_ref)
    t_ref[...] = jnp.dot(h, wl2_ref[...], preferred_element_type=jnp.float32)
    x_ref[...] = jnp.dot(h, wr2_ref[...], preferred_element_type=jnp.float32)


def _combine1(S, cnt, h, wful, bl, wl2, wr2):
    return pl.pallas_call(
        _combine1_body,
        grid=(G,),
        in_specs=[
            pl.BlockSpec((2, R, 32), lambda i: (0, i, 0)),
            pl.BlockSpec((R, CW), lambda i: (i, 0)),
            pl.BlockSpec((2, R, 32), lambda i: (0, i, 0)),
            pl.BlockSpec((2 * DH, DH), lambda i: (0, 0)),
            pl.BlockSpec((1, DH), lambda i: (0, 0)),
            pl.BlockSpec((DH, DO), lambda i: (0, 0)),
            pl.BlockSpec((DH, DO), lambda i: (0, 0)),
        ],
        out_specs=[
            pl.BlockSpec((R, DO), lambda i: (i, 0)),
            pl.BlockSpec((R, DO), lambda i: (i, 0)),
        ],
        out_shape=[
            jax.ShapeDtypeStruct((N, DO), jnp.float32),
            jax.ShapeDtypeStruct((N, DO), jnp.float32),
        ],
    )(S, cnt, h, wful, bl.reshape(1, DH), wl2, wr2)


def _layer2_core(s_ref, c_ref, x_ref, b_ref):
    cnt = c_ref[...]
    inv = 1.0 / jnp.maximum(cnt[:, 0:1], 1.0)
    h = (s_ref[0] + s_ref[1]) * inv + x_ref[...] + b_ref[...]
    return _leaky(_safe(h))


def _combine2_body(s_ref, c_ref, x_ref, b_ref, o_ref):
    o_ref[...] = _layer2_core(s_ref, c_ref, x_ref, b_ref)


def _combine2(S, cnt, xr, bl):
    return pl.pallas_call(
        _combine2_body,
        grid=(G,),
        in_specs=[
            pl.BlockSpec((2, R, 32), lambda i: (0, i, 0)),
            pl.BlockSpec((R, CW), lambda i: (i, 0)),
            pl.BlockSpec((R, DO), lambda i: (i, 0)),
            pl.BlockSpec((1, DO), lambda i: (0, 0)),
        ],
        out_specs=pl.BlockSpec((R, DO), lambda i: (i, 0)),
        out_shape=jax.ShapeDtypeStruct((N, DO), jnp.float32),
    )(S, cnt, xr, bl.reshape(1, DO))


def _user_body(s_ref, c_ref, x_ref, b_ref, w1_ref, b1_ref, w2_ref,
               b2_ref, o_ref, p_ref):
    h = _layer2_core(s_ref, c_ref, x_ref, b_ref)
    o_ref[...] = h
    z = _leaky(jnp.dot(h, w1_ref[...], preferred_element_type=jnp.float32)
               + b1_ref[...])
    p = jnp.dot(z, w2_ref[...], preferred_element_type=jnp.float32) + b2_ref[...]
    p_ref[...] = jax.nn.sigmoid(p)


def _combine2_user(S, cnt, xr, bl, w1, b1, w2, b2):
    return pl.pallas_call(
        _user_body,
        grid=(G,),
        in_specs=[
            pl.BlockSpec((2, R, 32), lambda i: (0, i, 0)),
            pl.BlockSpec((R, CW), lambda i: (i, 0)),
            pl.BlockSpec((R, DO), lambda i: (i, 0)),
            pl.BlockSpec((1, DO), lambda i: (0, 0)),
            pl.BlockSpec((DO, 16), lambda i: (0, 0)),
            pl.BlockSpec((1, 16), lambda i: (0, 0)),
            pl.BlockSpec((16, 1), lambda i: (0, 0)),
            pl.BlockSpec((1, 1), lambda i: (0, 0)),
        ],
        out_specs=[
            pl.BlockSpec((R, DO), lambda i: (i, 0)),
            pl.BlockSpec((R, 1), lambda i: (i, 0)),
        ],
        out_shape=[
            jax.ShapeDtypeStruct((N, DO), jnp.float32),
            jax.ShapeDtypeStruct((N, 1), jnp.float32),
        ],
    )(S, cnt, xr, bl.reshape(1, DO), w1, b1.reshape(1, 16), w2,
      b2.reshape(1, 1))


def _edge_arrays(edge_index):
    ei = edge_index.astype(jnp.int32)
    pad_i = jnp.arange(PAD, dtype=jnp.int32)
    src = jnp.concatenate([ei[0], pad_i % N])
    dst = jnp.concatenate([ei[1], N + pad_i % (NACC - N)])
    src2 = jnp.concatenate([src, src + N]).reshape(2 * ER, 128)
    return src2, src.reshape(ER, 128), dst.reshape(ER, 128), dst


def kernel(x_user, x_item, edge_index_ui, edge_index_iu,
           proj_user_W, proj_user_b, proj_item_W, proj_item_b,
           conv1_ui_Wl, conv1_ui_bl, conv1_ui_Wr,
           conv1_iu_Wl, conv1_iu_bl, conv1_iu_Wr,
           conv2_ui_Wl, conv2_ui_bl, conv2_ui_Wr,
           conv2_iu_Wl, conv2_iu_bl, conv2_iu_Wr,
           pred_W1, pred_b1, pred_W2, pred_b2):
    src2_ui, srcr_ui, dst2_ui, dstp_ui = _edge_arrays(edge_index_ui)
    src2_iu, srcr_iu, dst2_iu, dstp_iu = _edge_arrays(edge_index_iu)
    dst_all = jnp.concatenate([dstp_ui, dstp_iu]).reshape(2 * ER, 128)
    zblk = jnp.zeros((RPT, 32), jnp.float32)
    zblkc = jnp.zeros((RPT, CW), jnp.float32)
    oblk = jnp.ones((128, CW), jnp.float32)

    counts = _sc_counts(dst_all, oblk, zblkc).reshape(2, NACC, CW)
    cnt_item = counts[0]
    cnt_user = counts[1]

    hu = _proj(x_user, proj_user_W, proj_user_b)
    hi = _proj(x_item, proj_item_W, proj_item_b)

    s_ui = _sc_segsum(hu.reshape(2 * N, 32), src2_ui, dst2_ui, zblk)
    s_iu = _sc_segsum(hi.reshape(2 * N, 32), src2_iu, dst2_iu, zblk)

    w1_ui = jnp.concatenate([conv1_ui_Wl, conv1_ui_Wr], axis=0)
    w1_iu = jnp.concatenate([conv1_iu_Wl, conv1_iu_Wr], axis=0)
    t2i, xri = _combine1(s_ui.reshape(2, NACC, 32), cnt_item, hi, w1_ui,
                         conv1_ui_bl, conv2_iu_Wl, conv2_ui_Wr)
    t2u, xru = _combine1(s_iu.reshape(2, NACC, 32), cnt_user, hu, w1_iu,
                         conv1_iu_bl, conv2_ui_Wl, conv2_iu_Wr)

    s2_ui = _sc_segsum_e(t2u, srcr_ui, dst2_ui, zblk).reshape(2, NACC, 32)
    s2_iu = _sc_segsum_e(t2i, srcr_iu, dst2_iu, zblk).reshape(2, NACC, 32)

    hi2 = _combine2(s2_ui, cnt_item, xri, conv2_ui_bl)
    hu2, pred = _combine2_user(s2_iu, cnt_user, xru, conv2_iu_bl,
                               pred_W1, pred_b1, pred_W2, pred_b2)
    return pred[:, 0], {"user": hu2, "item": hi2}
